# Initial kernel scaffold; baseline (speedup 1.0000x reference)
import jax
import jax.numpy as jnp
from jax.experimental import pallas as pl


def kernel(x, edge_index, batch, edge_weight, W1, b1, g1, be1, W2, b2, g2, be2, W3, b3, g3, be3, lw1, lb1, lw2, lb2):
    raise NotImplementedError("write your pallas kernel here")


# speedup vs baseline: 2.7144x; 2.7144x over previous
import functools

import jax
import jax.numpy as jnp
from jax import lax
from jax.experimental import pallas as pl
from jax.experimental.pallas import tpu as pltpu

_N, _E, _D, _H, _G = 10000, 320000, 128, 128, 64
_SQRT2 = 1.4142135623730951


def _gelu(x):
    return 0.5 * x * (1.0 + lax.erf(x / _SQRT2))


def _bn_gelu(gcn, g, be):
    m = jnp.mean(gcn, axis=0, keepdims=True)
    v = jnp.mean((gcn - m) ** 2, axis=0, keepdims=True)
    return _gelu((gcn - m) * lax.rsqrt(v + 1e-5) * g + be)


def _pre_body(x_ref, w_ref, dinv_ref, y_ref):
    y_ref[...] = dinv_ref[...] * jnp.dot(
        x_ref[...], w_ref[...], preferred_element_type=jnp.float32)


def _mid_body(agg_ref, y_ref, dinv_ref, b_ref, g_ref, be_ref, w_ref, out_ref):
    dinv = dinv_ref[...]
    gcn = dinv * (agg_ref[...] + y_ref[...]) + b_ref[...]
    h = _bn_gelu(gcn, g_ref[...], be_ref[...])
    out_ref[...] = dinv * jnp.dot(h, w_ref[...],
                                  preferred_element_type=jnp.float32)


def _fin_body(agg_ref, y_ref, dinv_ref, b_ref, g_ref, be_ref, batch_ref,
              lw1_ref, lb1_ref, lw2_ref, lb2_ref, out_ref):
    dinv = dinv_ref[...]
    gcn = dinv * (agg_ref[...] + y_ref[...]) + b_ref[...]
    h = _bn_gelu(gcn, g_ref[...], be_ref[...])
    gids = lax.broadcasted_iota(jnp.int32, (_G, _N), 0)
    onehot = (batch_ref[...] == gids).astype(jnp.float32)
    s = jnp.dot(onehot, h, preferred_element_type=jnp.float32)
    cnt = jnp.sum(onehot, axis=1, keepdims=True)
    pooled = s / jnp.maximum(cnt, 1.0)
    o = pooled @ lw1_ref[...] + lb1_ref[...]
    o = jnp.where(o > 0, o, jnp.exp(jnp.minimum(o, 0.0)) - 1.0)
    out_ref[...] = o @ lw2_ref[...] + lb2_ref[...]


def _aggregate(y, src, dst):
    return jax.ops.segment_sum(y[src], dst, num_segments=_N)


def _degree(dst):
    return jax.ops.segment_sum(jnp.ones((_E,), jnp.float32), dst,
                               num_segments=_N) + 1.0


def kernel(x, edge_index, batch, edge_weight, W1, b1, g1, be1, W2, b2, g2,
           be2, W3, b3, g3, be3, lw1, lb1, lw2, lb2):
    src = edge_index[0]
    dst = edge_index[1]
    deg = _degree(dst)
    dinv = lax.rsqrt(deg).reshape(_N, 1)

    pre = pl.pallas_call(
        _pre_body, out_shape=jax.ShapeDtypeStruct((_N, _H), jnp.float32))
    mid = pl.pallas_call(
        _mid_body, out_shape=jax.ShapeDtypeStruct((_N, _H), jnp.float32))
    fin = pl.pallas_call(
        _fin_body, out_shape=jax.ShapeDtypeStruct((_G, 1), jnp.float32))

    y1 = pre(x, W1, dinv)
    agg1 = _aggregate(y1, src, dst)
    y2 = mid(agg1, y1, dinv, b1.reshape(1, _H), g1.reshape(1, _H),
             be1.reshape(1, _H), W2)
    agg2 = _aggregate(y2, src, dst)
    y3 = mid(agg2, y2, dinv, b2.reshape(1, _H), g2.reshape(1, _H),
             be2.reshape(1, _H), W3)
    agg3 = _aggregate(y3, src, dst)
    out = fin(agg3, y3, dinv, b3.reshape(1, _H), g3.reshape(1, _H),
              be3.reshape(1, _H), batch.reshape(1, _N),
              lw1, lb1.reshape(1, _H // 2), lw2, lb2.reshape(1, 1))
    return out


# speedup vs baseline: 22.7206x; 8.3704x over previous
import functools

import jax
import jax.numpy as jnp
from jax import lax
from jax.experimental import pallas as pl
from jax.experimental.pallas import tpu as pltpu
from jax.experimental.pallas import tpu_sc as plsc

_N, _E, _D, _H, _G = 10000, 320000, 128, 128, 64
_SQRT2 = 1.4142135623730951

_NC, _NS = 2, 16
_NW = _NC * _NS
_K = 80
_NCH = _E // (_NW * _K)
_RPT = _N // _NS

@functools.cache
def _sc_mesh():
    return plsc.VectorSubcoreMesh(
        core_axis_name="c", subcore_axis_name="s",
        num_cores=_NC, num_subcores=_NS)


def _deg_body(dst_hbm, zero_hbm, out_hbm, dstv, onesv, acc):
    c = lax.axis_index("c")
    s = lax.axis_index("s")
    wid = c * _NS + s
    pltpu.sync_copy(dst_hbm.at[pl.ds(wid * _NCH, _NCH)], dstv)
    pltpu.sync_copy(zero_hbm.at[pl.ds(s * _RPT, _RPT)],
                    acc.at[pl.ds(s * _RPT, _RPT)])

    def fill(i, carry):
        onesv[i, :] = jnp.ones((16,), jnp.float32)
        return carry

    lax.fori_loop(0, _K, fill, 0)
    plsc.subcore_barrier()

    def body(i, carry):
        pltpu.sync_copy(onesv, acc.at[dstv.at[i]], add=True)
        return carry

    lax.fori_loop(0, _NCH, body, 0)
    plsc.subcore_barrier()
    pltpu.sync_copy(acc.at[pl.ds(s * _RPT, _RPT)],
                    out_hbm.at[c].at[pl.ds(s * _RPT, _RPT)])


@jax.jit
def _sc_degree(dst2d, zeros16):
    return pl.kernel(
        _deg_body,
        out_type=jax.ShapeDtypeStruct((_NC, _N, 16), jnp.float32),
        mesh=_sc_mesh(),
        scratch_types=[
            pltpu.VMEM((_NCH, _K), jnp.int32),
            pltpu.VMEM((_K, 16), jnp.float32),
            pltpu.VMEM_SHARED((_N, 16), jnp.float32),
        ],
        compiler_params=pltpu.CompilerParams(use_tc_tiling_on_sc=False),
    )(dst2d, zeros16)


def _agg_body(y_hbm, src_hbm, dst_hbm, zero_hbm, out_hbm,
              srcv, dstv, rows0, rows1, acc, gsem, ssem):
    c = lax.axis_index("c")
    s = lax.axis_index("s")
    wid = c * _NS + s
    pltpu.sync_copy(src_hbm.at[pl.ds(wid * _NCH, _NCH)], srcv)
    pltpu.sync_copy(dst_hbm.at[pl.ds(wid * _NCH, _NCH)], dstv)
    pltpu.sync_copy(zero_hbm.at[pl.ds(s * _RPT, _RPT)],
                    acc.at[pl.ds(s * _RPT, _RPT)])
    plsc.subcore_barrier()

    def body(j, carry):
        i0 = 2 * j
        i1 = 2 * j + 1
        g0 = pltpu.async_copy(y_hbm.at[srcv.at[i0]], rows0, gsem)
        g1 = pltpu.async_copy(y_hbm.at[srcv.at[i1]], rows1, gsem)
        g0.wait()
        s0 = pltpu.async_copy(rows0, acc.at[dstv.at[i0]], ssem, add=True)
        g1.wait()
        s1 = pltpu.async_copy(rows1, acc.at[dstv.at[i1]], ssem, add=True)
        s0.wait()
        s1.wait()
        return carry

    lax.fori_loop(0, _NCH // 2, body, 0)
    glast = pltpu.async_copy(y_hbm.at[srcv.at[_NCH - 1]], rows0, gsem)
    glast.wait()
    pltpu.sync_copy(rows0, acc.at[dstv.at[_NCH - 1]], add=True)

    plsc.subcore_barrier()
    pltpu.sync_copy(acc.at[pl.ds(s * _RPT, _RPT)],
                    out_hbm.at[c].at[pl.ds(s * _RPT, _RPT)])


@jax.jit
def _sc_aggregate(y, src2d, dst2d, zeros):
    return pl.kernel(
        _agg_body,
        out_type=jax.ShapeDtypeStruct((_NC, _N, _H), jnp.float32),
        mesh=_sc_mesh(),
        scratch_types=[
            pltpu.VMEM((_NCH, _K), jnp.int32),
            pltpu.VMEM((_NCH, _K), jnp.int32),
            pltpu.VMEM((_K, _H), jnp.float32),
            pltpu.VMEM((_K, _H), jnp.float32),
            pltpu.VMEM_SHARED((_N, _H), jnp.float32),
            pltpu.SemaphoreType.DMA,
            pltpu.SemaphoreType.DMA,
        ],
        compiler_params=pltpu.CompilerParams(use_tc_tiling_on_sc=False),
    )(y, src2d, dst2d, zeros)


def _gelu(x):
    return 0.5 * x * (1.0 + lax.erf(x / _SQRT2))


def _bn_gelu(gcn, g, be):
    m = jnp.mean(gcn, axis=0, keepdims=True)
    v = jnp.mean((gcn - m) ** 2, axis=0, keepdims=True)
    return _gelu((gcn - m) * lax.rsqrt(v + 1e-5) * g + be)


def _matmul_body(x_ref, w_ref, o_ref):
    o_ref[...] = jnp.dot(x_ref[...], w_ref[...],
                         preferred_element_type=jnp.float32)


def _scale_body(xw_ref, degp_ref, y_ref, dinv_ref):
    deg = degp_ref[0, :, :1] + degp_ref[1, :, :1] + 1.0
    dinv = lax.rsqrt(deg)
    dinv_ref[...] = dinv
    y_ref[...] = dinv * xw_ref[...]


def _mid_body(aggp_ref, y_ref, dinv_ref, b_ref, g_ref, be_ref, w_ref,
              out_ref):
    dinv = dinv_ref[...]
    gcn = dinv * (aggp_ref[0] + aggp_ref[1] + y_ref[...]) + b_ref[...]
    h = _bn_gelu(gcn, g_ref[...], be_ref[...])
    out_ref[...] = dinv * jnp.dot(h, w_ref[...],
                                  preferred_element_type=jnp.float32)


def _fin_body(aggp_ref, y_ref, dinv_ref, b_ref, g_ref, be_ref, batch_ref,
              lw1_ref, lb1_ref, lw2_ref, lb2_ref, out_ref):
    dinv = dinv_ref[...]
    gcn = dinv * (aggp_ref[0] + aggp_ref[1] + y_ref[...]) + b_ref[...]
    h = _bn_gelu(gcn, g_ref[...], be_ref[...])
    gids = lax.broadcasted_iota(jnp.int32, (_G, _N), 0)
    onehot = (batch_ref[...] == gids).astype(jnp.float32)
    sums = jnp.dot(onehot, h, preferred_element_type=jnp.float32)
    cnt = jnp.sum(onehot, axis=1, keepdims=True)
    pooled = sums / jnp.maximum(cnt, 1.0)
    o = pooled @ lw1_ref[...] + lb1_ref[...]
    o = jnp.where(o > 0, o, jnp.exp(jnp.minimum(o, 0.0)) - 1.0)
    out_ref[...] = o @ lw2_ref[...] + lb2_ref[...]


def kernel(x, edge_index, batch, edge_weight, W1, b1, g1, be1, W2, b2, g2,
           be2, W3, b3, g3, be3, lw1, lb1, lw2, lb2):
    src2d = edge_index[0].reshape(_E // _K, _K)
    dst2d = edge_index[1].reshape(_E // _K, _K)
    zeros = jnp.zeros((_N, _H), jnp.float32)
    zeros16 = jnp.zeros((_N, 16), jnp.float32)

    matmul = pl.pallas_call(
        _matmul_body, out_shape=jax.ShapeDtypeStruct((_N, _H), jnp.float32))
    scale = pl.pallas_call(
        _scale_body, out_shape=[jax.ShapeDtypeStruct((_N, _H), jnp.float32),
                                jax.ShapeDtypeStruct((_N, 1), jnp.float32)])
    mid = pl.pallas_call(
        _mid_body, out_shape=jax.ShapeDtypeStruct((_N, _H), jnp.float32))
    fin = pl.pallas_call(
        _fin_body, out_shape=jax.ShapeDtypeStruct((_G, 1), jnp.float32))

    degp = _sc_degree(dst2d, zeros16)
    xw1 = matmul(x, W1)
    y1, dinv = scale(xw1, degp)

    agg1 = _sc_aggregate(y1, src2d, dst2d, zeros)
    y2 = mid(agg1, y1, dinv, b1.reshape(1, _H), g1.reshape(1, _H),
             be1.reshape(1, _H), W2)
    agg2 = _sc_aggregate(y2, src2d, dst2d, zeros)
    y3 = mid(agg2, y2, dinv, b2.reshape(1, _H), g2.reshape(1, _H),
             be2.reshape(1, _H), W3)
    agg3 = _sc_aggregate(y3, src2d, dst2d, zeros)
    out = fin(agg3, y3, dinv, b3.reshape(1, _H), g3.reshape(1, _H),
              be3.reshape(1, _H), batch.reshape(1, _N),
              lw1, lb1.reshape(1, _H // 2), lw2, lb2.reshape(1, 1))
    return out


# speedup vs baseline: 30.6798x; 1.3503x over previous
import functools

import jax
import jax.numpy as jnp
from jax import lax
from jax.experimental import pallas as pl
from jax.experimental.pallas import tpu as pltpu
from jax.experimental.pallas import tpu_sc as plsc

_N, _E, _D, _H, _G = 10000, 320000, 128, 128, 64
_SQRT2 = 1.4142135623730951

_NC, _NS = 2, 16
_NW = _NC * _NS
_K = 40
_NCH = _E // (_NW * _K)
_RPT = _N // _NS

@functools.cache
def _sc_mesh():
    return plsc.VectorSubcoreMesh(
        core_axis_name="c", subcore_axis_name="s",
        num_cores=_NC, num_subcores=_NS)


def _deg_body(dst_hbm, zero_hbm, out_hbm, dstv, onesv, acc):
    c = lax.axis_index("c")
    s = lax.axis_index("s")
    wid = c * _NS + s
    pltpu.sync_copy(dst_hbm.at[pl.ds(wid * _NCH, _NCH)], dstv)
    pltpu.sync_copy(zero_hbm.at[pl.ds(s * _RPT, _RPT)],
                    acc.at[pl.ds(s * _RPT, _RPT)])

    def fill(i, carry):
        onesv[i, :] = jnp.ones((16,), jnp.float32)
        return carry

    lax.fori_loop(0, _K, fill, 0)
    plsc.subcore_barrier()

    def body(i, carry):
        pltpu.sync_copy(onesv, acc.at[dstv.at[i]], add=True)
        return carry

    lax.fori_loop(0, _NCH, body, 0)
    plsc.subcore_barrier()
    pltpu.sync_copy(acc.at[pl.ds(s * _RPT, _RPT)],
                    out_hbm.at[c].at[pl.ds(s * _RPT, _RPT)])


@jax.jit
def _sc_degree(dst2d, zeros16):
    return pl.kernel(
        _deg_body,
        out_type=jax.ShapeDtypeStruct((_NC, _N, 16), jnp.float32),
        mesh=_sc_mesh(),
        scratch_types=[
            pltpu.VMEM((_NCH, _K), jnp.int32),
            pltpu.VMEM((_K, 16), jnp.float32),
            pltpu.VMEM_SHARED((_N, 16), jnp.float32),
        ],
        compiler_params=pltpu.CompilerParams(use_tc_tiling_on_sc=False),
    )(dst2d, zeros16)


_NB = 5


def _agg_body(y_hbm, src_hbm, dst_hbm, zero_hbm, out_hbm,
              srcv, dstv, r0, r1, r2, r3, r4,
              acc, gs0, gs1, gs2, gs3, gs4, ss0, ss1, ss2, ss3, ss4):
    rows = [r0, r1, r2, r3, r4]
    gsem = [gs0, gs1, gs2, gs3, gs4]
    ssem = [ss0, ss1, ss2, ss3, ss4]
    c = lax.axis_index("c")
    s = lax.axis_index("s")
    wid = c * _NS + s
    pltpu.sync_copy(src_hbm.at[pl.ds(wid * _NCH, _NCH)], srcv)
    pltpu.sync_copy(dst_hbm.at[pl.ds(wid * _NCH, _NCH)], dstv)
    pltpu.sync_copy(zero_hbm.at[pl.ds(s * _RPT, _RPT)],
                    acc.at[pl.ds(s * _RPT, _RPT)])
    for b in range(_NB):
        pltpu.async_copy(y_hbm.at[srcv.at[b]], rows[b], gsem[b])
    plsc.subcore_barrier()

    def body(g, carry):
        for b in range(_NB):
            i = g * _NB + b
            pltpu.make_async_copy(y_hbm.at[srcv.at[i]], rows[b],
                                  gsem[b]).wait()
            pltpu.async_copy(rows[b], acc.at[dstv.at[i]], ssem[b], add=True)
            pltpu.make_async_copy(rows[b], acc.at[dstv.at[i]],
                                  ssem[b]).wait()
            pltpu.async_copy(y_hbm.at[srcv.at[i + _NB]], rows[b], gsem[b])
        return carry

    lax.fori_loop(0, _NCH // _NB - 1, body, 0)
    scats = []
    for b in range(_NB):
        i = _NCH - _NB + b
        pltpu.make_async_copy(y_hbm.at[srcv.at[i]], rows[b], gsem[b]).wait()
        scats.append(
            pltpu.async_copy(rows[b], acc.at[dstv.at[i]], ssem[b], add=True))
    for cp in scats:
        cp.wait()

    plsc.subcore_barrier()
    pltpu.sync_copy(acc.at[pl.ds(s * _RPT, _RPT)],
                    out_hbm.at[c].at[pl.ds(s * _RPT, _RPT)])


@jax.jit
def _sc_aggregate(y, src2d, dst2d, zeros):
    return pl.kernel(
        _agg_body,
        out_type=jax.ShapeDtypeStruct((_NC, _N, _H), jnp.float32),
        mesh=_sc_mesh(),
        scratch_types=[
            pltpu.VMEM((_NCH, _K), jnp.int32),
            pltpu.VMEM((_NCH, _K), jnp.int32),
        ] + [pltpu.VMEM((_K, _H), jnp.float32)] * _NB + [
            pltpu.VMEM_SHARED((_N, _H), jnp.float32),
        ] + [pltpu.SemaphoreType.DMA] * (2 * _NB),
        compiler_params=pltpu.CompilerParams(use_tc_tiling_on_sc=False),
    )(y, src2d, dst2d, zeros)


def _gelu(x):
    return 0.5 * x * (1.0 + lax.erf(x / _SQRT2))


def _bn_gelu(gcn, g, be):
    m = jnp.mean(gcn, axis=0, keepdims=True)
    v = jnp.mean((gcn - m) ** 2, axis=0, keepdims=True)
    return _gelu((gcn - m) * lax.rsqrt(v + 1e-5) * g + be)


def _matmul_body(x_ref, w_ref, o_ref):
    o_ref[...] = jnp.dot(x_ref[...], w_ref[...],
                         preferred_element_type=jnp.float32)


def _scale_body(xw_ref, degp_ref, y_ref, dinv_ref):
    deg = degp_ref[0, :, :1] + degp_ref[1, :, :1] + 1.0
    dinv = lax.rsqrt(deg)
    dinv_ref[...] = dinv
    y_ref[...] = dinv * xw_ref[...]


def _mid_body(aggp_ref, y_ref, dinv_ref, b_ref, g_ref, be_ref, w_ref,
              out_ref):
    dinv = dinv_ref[...]
    gcn = dinv * (aggp_ref[0] + aggp_ref[1] + y_ref[...]) + b_ref[...]
    h = _bn_gelu(gcn, g_ref[...], be_ref[...])
    out_ref[...] = dinv * jnp.dot(h, w_ref[...],
                                  preferred_element_type=jnp.float32)


def _fin_body(aggp_ref, y_ref, dinv_ref, b_ref, g_ref, be_ref, batch_ref,
              lw1_ref, lb1_ref, lw2_ref, lb2_ref, out_ref):
    dinv = dinv_ref[...]
    gcn = dinv * (aggp_ref[0] + aggp_ref[1] + y_ref[...]) + b_ref[...]
    h = _bn_gelu(gcn, g_ref[...], be_ref[...])
    gids = lax.broadcasted_iota(jnp.int32, (_G, _N), 0)
    onehot = (batch_ref[...] == gids).astype(jnp.float32)
    sums = jnp.dot(onehot, h, preferred_element_type=jnp.float32)
    cnt = jnp.sum(onehot, axis=1, keepdims=True)
    pooled = sums / jnp.maximum(cnt, 1.0)
    o = pooled @ lw1_ref[...] + lb1_ref[...]
    o = jnp.where(o > 0, o, jnp.exp(jnp.minimum(o, 0.0)) - 1.0)
    out_ref[...] = o @ lw2_ref[...] + lb2_ref[...]


def kernel(x, edge_index, batch, edge_weight, W1, b1, g1, be1, W2, b2, g2,
           be2, W3, b3, g3, be3, lw1, lb1, lw2, lb2):
    src2d = edge_index[0].reshape(_E // _K, _K)
    dst2d = edge_index[1].reshape(_E // _K, _K)
    zeros = jnp.zeros((_N, _H), jnp.float32)
    zeros16 = jnp.zeros((_N, 16), jnp.float32)

    matmul = pl.pallas_call(
        _matmul_body, out_shape=jax.ShapeDtypeStruct((_N, _H), jnp.float32))
    scale = pl.pallas_call(
        _scale_body, out_shape=[jax.ShapeDtypeStruct((_N, _H), jnp.float32),
                                jax.ShapeDtypeStruct((_N, 1), jnp.float32)])
    mid = pl.pallas_call(
        _mid_body, out_shape=jax.ShapeDtypeStruct((_N, _H), jnp.float32))
    fin = pl.pallas_call(
        _fin_body, out_shape=jax.ShapeDtypeStruct((_G, 1), jnp.float32))

    degp = _sc_degree(dst2d, zeros16)
    xw1 = matmul(x, W1)
    y1, dinv = scale(xw1, degp)

    agg1 = _sc_aggregate(y1, src2d, dst2d, zeros)
    y2 = mid(agg1, y1, dinv, b1.reshape(1, _H), g1.reshape(1, _H),
             be1.reshape(1, _H), W2)
    agg2 = _sc_aggregate(y2, src2d, dst2d, zeros)
    y3 = mid(agg2, y2, dinv, b2.reshape(1, _H), g2.reshape(1, _H),
             be2.reshape(1, _H), W3)
    agg3 = _sc_aggregate(y3, src2d, dst2d, zeros)
    out = fin(agg3, y3, dinv, b3.reshape(1, _H), g3.reshape(1, _H),
              be3.reshape(1, _H), batch.reshape(1, _N),
              lw1, lb1.reshape(1, _H // 2), lw2, lb2.reshape(1, 1))
    return out


# speedup vs baseline: 32.5397x; 1.0606x over previous
import functools

import jax
import jax.numpy as jnp
from jax import lax
from jax.experimental import pallas as pl
from jax.experimental.pallas import tpu as pltpu
from jax.experimental.pallas import tpu_sc as plsc

_N, _E, _D, _H, _G = 10000, 320000, 128, 128, 64
_SQRT2 = 1.4142135623730951

_NC, _NS = 2, 16
_NW = _NC * _NS
_RPT = _N // _NS

_K = 40
_NCH = _E // (_NW * _K)
_NB = 5
_ZR = 25

_DK = 80
_DNCH = _E // (_NW * _DK)


@functools.cache
def _sc_mesh():
    return plsc.VectorSubcoreMesh(
        core_axis_name="c", subcore_axis_name="s",
        num_cores=_NC, num_subcores=_NS)


def _memset_zero(zblk, acc, s, width):
    def fill(i, carry):
        def fill_col(j, carry2):
            zblk[i, pl.ds(j * 16, 16)] = jnp.zeros((16,), jnp.float32)
            return carry2

        return lax.fori_loop(0, width // 16, fill_col, carry)

    lax.fori_loop(0, _ZR, fill, 0)

    def cp(j, carry):
        pltpu.sync_copy(zblk, acc.at[pl.ds(s * _RPT + j * _ZR, _ZR)])
        return carry

    lax.fori_loop(0, _RPT // _ZR, cp, 0)


def _deg_body(dst_hbm, out_hbm, dstv, onesv, zblk, acc):
    c = lax.axis_index("c")
    s = lax.axis_index("s")
    wid = c * _NS + s
    pltpu.sync_copy(dst_hbm.at[pl.ds(wid * _DNCH, _DNCH)], dstv)

    def fill(i, carry):
        onesv[i, :] = jnp.ones((16,), jnp.float32)
        return carry

    lax.fori_loop(0, _DK, fill, 0)
    _memset_zero(zblk, acc, s, 16)
    plsc.subcore_barrier()

    def body(i, carry):
        pltpu.sync_copy(onesv, acc.at[dstv.at[i]], add=True)
        return carry

    lax.fori_loop(0, _DNCH, body, 0)
    plsc.subcore_barrier()
    pltpu.sync_copy(acc.at[pl.ds(s * _RPT, _RPT)],
                    out_hbm.at[c].at[pl.ds(s * _RPT, _RPT)])


@jax.jit
def _sc_degree(dst2d):
    return pl.kernel(
        _deg_body,
        out_type=jax.ShapeDtypeStruct((_NC, _N, 16), jnp.float32),
        mesh=_sc_mesh(),
        scratch_types=[
            pltpu.VMEM((_DNCH, _DK), jnp.int32),
            pltpu.VMEM((_DK, 16), jnp.float32),
            pltpu.VMEM((_ZR, 16), jnp.float32),
            pltpu.VMEM_SHARED((_N, 16), jnp.float32),
        ],
        compiler_params=pltpu.CompilerParams(use_tc_tiling_on_sc=False),
    )(dst2d)


def _agg_body(y_hbm, src_hbm, dst_hbm, out_hbm,
              srcv, dstv, r0, r1, r2, r3, r4, zblk,
              acc, gs0, gs1, gs2, gs3, gs4, ss0, ss1, ss2, ss3, ss4):
    rows = [r0, r1, r2, r3, r4]
    gsem = [gs0, gs1, gs2, gs3, gs4]
    ssem = [ss0, ss1, ss2, ss3, ss4]
    c = lax.axis_index("c")
    s = lax.axis_index("s")
    wid = c * _NS + s
    pltpu.sync_copy(src_hbm.at[pl.ds(wid * _NCH, _NCH)], srcv)
    pltpu.sync_copy(dst_hbm.at[pl.ds(wid * _NCH, _NCH)], dstv)
    for b in range(_NB):
        pltpu.async_copy(y_hbm.at[srcv.at[b]], rows[b], gsem[b])
    _memset_zero(zblk, acc, s, _H)
    plsc.subcore_barrier()

    def body(g, carry):
        for b in range(_NB):
            i = g * _NB + b
            pltpu.make_async_copy(y_hbm.at[srcv.at[i]], rows[b],
                                  gsem[b]).wait()
            pltpu.async_copy(rows[b], acc.at[dstv.at[i]], ssem[b], add=True)
            pltpu.make_async_copy(rows[b], acc.at[dstv.at[i]],
                                  ssem[b]).wait()
            pltpu.async_copy(y_hbm.at[srcv.at[i + _NB]], rows[b], gsem[b])
        return carry

    lax.fori_loop(0, _NCH // _NB - 1, body, 0)
    scats = []
    for b in range(_NB):
        i = _NCH - _NB + b
        pltpu.make_async_copy(y_hbm.at[srcv.at[i]], rows[b], gsem[b]).wait()
        scats.append(
            pltpu.async_copy(rows[b], acc.at[dstv.at[i]], ssem[b], add=True))
    for cp in scats:
        cp.wait()

    plsc.subcore_barrier()
    pltpu.sync_copy(acc.at[pl.ds(s * _RPT, _RPT)],
                    out_hbm.at[c].at[pl.ds(s * _RPT, _RPT)])


@jax.jit
def _sc_aggregate(y, src2d, dst2d):
    return pl.kernel(
        _agg_body,
        out_type=jax.ShapeDtypeStruct((_NC, _N, _H), jnp.float32),
        mesh=_sc_mesh(),
        scratch_types=[
            pltpu.VMEM((_NCH, _K), jnp.int32),
            pltpu.VMEM((_NCH, _K), jnp.int32),
        ] + [pltpu.VMEM((_K, _H), jnp.float32)] * _NB + [
            pltpu.VMEM((_ZR, _H), jnp.float32),
            pltpu.VMEM_SHARED((_N, _H), jnp.float32),
        ] + [pltpu.SemaphoreType.DMA] * (2 * _NB),
        compiler_params=pltpu.CompilerParams(use_tc_tiling_on_sc=False),
    )(y, src2d, dst2d)


def _gelu(x):
    return 0.5 * x * (1.0 + lax.erf(x / _SQRT2))


def _bn_gelu(gcn, g, be):
    m = jnp.mean(gcn, axis=0, keepdims=True)
    v = jnp.mean((gcn - m) ** 2, axis=0, keepdims=True)
    return _gelu((gcn - m) * lax.rsqrt(v + 1e-5) * g + be)


def _matmul_body(x_ref, w_ref, o_ref):
    o_ref[...] = jnp.dot(x_ref[...], w_ref[...],
                         preferred_element_type=jnp.float32)


def _scale_body(xw_ref, degp_ref, y_ref, dinv_ref):
    deg = degp_ref[0, :, :1] + degp_ref[1, :, :1] + 1.0
    dinv = lax.rsqrt(deg)
    dinv_ref[...] = dinv
    y_ref[...] = dinv * xw_ref[...]


def _mid_body(aggp_ref, y_ref, dinv_ref, b_ref, g_ref, be_ref, w_ref,
              out_ref):
    dinv = dinv_ref[...]
    gcn = dinv * (aggp_ref[0] + aggp_ref[1] + y_ref[...]) + b_ref[...]
    h = _bn_gelu(gcn, g_ref[...], be_ref[...])
    out_ref[...] = dinv * jnp.dot(h, w_ref[...],
                                  preferred_element_type=jnp.float32)


def _fin_body(aggp_ref, y_ref, dinv_ref, b_ref, g_ref, be_ref, batch_ref,
              lw1_ref, lb1_ref, lw2_ref, lb2_ref, out_ref):
    dinv = dinv_ref[...]
    gcn = dinv * (aggp_ref[0] + aggp_ref[1] + y_ref[...]) + b_ref[...]
    h = _bn_gelu(gcn, g_ref[...], be_ref[...])
    gids = lax.broadcasted_iota(jnp.int32, (_G, _N), 0)
    onehot = (batch_ref[...] == gids).astype(jnp.float32)
    sums = jnp.dot(onehot, h, preferred_element_type=jnp.float32)
    cnt = jnp.sum(onehot, axis=1, keepdims=True)
    pooled = sums / jnp.maximum(cnt, 1.0)
    o = pooled @ lw1_ref[...] + lb1_ref[...]
    o = jnp.where(o > 0, o, jnp.exp(jnp.minimum(o, 0.0)) - 1.0)
    out_ref[...] = o @ lw2_ref[...] + lb2_ref[...]


def kernel(x, edge_index, batch, edge_weight, W1, b1, g1, be1, W2, b2, g2,
           be2, W3, b3, g3, be3, lw1, lb1, lw2, lb2):
    src2d = edge_index[0].reshape(_E // _K, _K)
    dst2d = edge_index[1].reshape(_E // _K, _K)
    dst2d_deg = edge_index[1].reshape(_E // _DK, _DK)

    matmul = pl.pallas_call(
        _matmul_body, out_shape=jax.ShapeDtypeStruct((_N, _H), jnp.float32))
    scale = pl.pallas_call(
        _scale_body, out_shape=[jax.ShapeDtypeStruct((_N, _H), jnp.float32),
                                jax.ShapeDtypeStruct((_N, 1), jnp.float32)])
    mid = pl.pallas_call(
        _mid_body, out_shape=jax.ShapeDtypeStruct((_N, _H), jnp.float32))
    fin = pl.pallas_call(
        _fin_body, out_shape=jax.ShapeDtypeStruct((_G, 1), jnp.float32))

    degp = _sc_degree(dst2d_deg)
    xw1 = matmul(x, W1)
    y1, dinv = scale(xw1, degp)

    agg1 = _sc_aggregate(y1, src2d, dst2d)
    y2 = mid(agg1, y1, dinv, b1.reshape(1, _H), g1.reshape(1, _H),
             be1.reshape(1, _H), W2)
    agg2 = _sc_aggregate(y2, src2d, dst2d)
    y3 = mid(agg2, y2, dinv, b2.reshape(1, _H), g2.reshape(1, _H),
             be2.reshape(1, _H), W3)
    agg3 = _sc_aggregate(y3, src2d, dst2d)
    out = fin(agg3, y3, dinv, b3.reshape(1, _H), g3.reshape(1, _H),
              be3.reshape(1, _H), batch.reshape(1, _N),
              lw1, lb1.reshape(1, _H // 2), lw2, lb2.reshape(1, 1))
    return out


# speedup vs baseline: 33.1954x; 1.0202x over previous
import functools

import jax
import jax.numpy as jnp
from jax import lax
from jax.experimental import pallas as pl
from jax.experimental.pallas import tpu as pltpu
from jax.experimental.pallas import tpu_sc as plsc

_N, _E, _D, _H, _G = 10000, 320000, 128, 128, 64
_SQRT2 = 1.4142135623730951

_NC, _NS = 2, 16
_NW = _NC * _NS
_RPT = _N // _NS

_K = 40
_NCH = _E // (_NW * _K)
_NB = 5
_ZR = 25

_DK = 80
_DNCH = _E // (_NW * _DK)


@functools.cache
def _sc_mesh():
    return plsc.VectorSubcoreMesh(
        core_axis_name="c", subcore_axis_name="s",
        num_cores=_NC, num_subcores=_NS)


def _memset_zero(zblk, acc, s, width):
    def fill(i, carry):
        def fill_col(j, carry2):
            zblk[i, pl.ds(j * 16, 16)] = jnp.zeros((16,), jnp.float32)
            return carry2

        return lax.fori_loop(0, width // 16, fill_col, carry)

    lax.fori_loop(0, _ZR, fill, 0)

    def cp(j, carry):
        pltpu.sync_copy(zblk, acc.at[pl.ds(s * _RPT + j * _ZR, _ZR)])
        return carry

    lax.fori_loop(0, _RPT // _ZR, cp, 0)


def _deg_body(dst_hbm, out_hbm, dstv, onesv, zblk, acc, sem):
    c = lax.axis_index("c")
    s = lax.axis_index("s")
    wid = c * _NS + s
    pltpu.sync_copy(dst_hbm.at[pl.ds(wid * _DNCH, _DNCH)], dstv)

    def fill(i, carry):
        onesv[i, :] = jnp.ones((16,), jnp.float32)
        return carry

    lax.fori_loop(0, _DK, fill, 0)
    _memset_zero(zblk, acc, s, 16)
    plsc.subcore_barrier()

    lag = 4
    for i in range(lag):
        pltpu.async_copy(onesv, acc.at[dstv.at[i]], sem, add=True)

    def body(i, carry):
        pltpu.async_copy(onesv, acc.at[dstv.at[i + lag]], sem, add=True)
        pltpu.make_async_copy(onesv, acc.at[dstv.at[i]], sem).wait()
        return carry

    lax.fori_loop(0, _DNCH - lag, body, 0)
    for i in range(_DNCH - lag, _DNCH):
        pltpu.make_async_copy(onesv, acc.at[dstv.at[i]], sem).wait()
    plsc.subcore_barrier()
    pltpu.sync_copy(acc.at[pl.ds(s * _RPT, _RPT)],
                    out_hbm.at[c].at[pl.ds(s * _RPT, _RPT)])


@jax.jit
def _sc_degree(dst2d):
    return pl.kernel(
        _deg_body,
        out_type=jax.ShapeDtypeStruct((_NC, _N, 16), jnp.float32),
        mesh=_sc_mesh(),
        scratch_types=[
            pltpu.VMEM((_DNCH, _DK), jnp.int32),
            pltpu.VMEM((_DK, 16), jnp.float32),
            pltpu.VMEM((_ZR, 16), jnp.float32),
            pltpu.VMEM_SHARED((_N, 16), jnp.float32),
            pltpu.SemaphoreType.DMA,
        ],
        compiler_params=pltpu.CompilerParams(use_tc_tiling_on_sc=False),
    )(dst2d)


def _agg_body(y_hbm, src_hbm, dst_hbm, out_hbm,
              srcv, dstv, r0, r1, r2, r3, r4, zblk,
              acc, gs0, gs1, gs2, gs3, gs4, ss0, ss1, ss2, ss3, ss4):
    rows = [r0, r1, r2, r3, r4]
    gsem = [gs0, gs1, gs2, gs3, gs4]
    ssem = [ss0, ss1, ss2, ss3, ss4]
    c = lax.axis_index("c")
    s = lax.axis_index("s")
    wid = c * _NS + s
    pltpu.sync_copy(src_hbm.at[pl.ds(wid * _NCH, _NCH)], srcv)
    pltpu.sync_copy(dst_hbm.at[pl.ds(wid * _NCH, _NCH)], dstv)
    for b in range(_NB):
        pltpu.async_copy(y_hbm.at[srcv.at[b]], rows[b], gsem[b])
    _memset_zero(zblk, acc, s, _H)
    plsc.subcore_barrier()

    def body(g, carry):
        for b in range(_NB):
            i = g * _NB + b
            pltpu.make_async_copy(y_hbm.at[srcv.at[i]], rows[b],
                                  gsem[b]).wait()
            pltpu.async_copy(rows[b], acc.at[dstv.at[i]], ssem[b], add=True)
            pltpu.make_async_copy(rows[b], acc.at[dstv.at[i]],
                                  ssem[b]).wait()
            pltpu.async_copy(y_hbm.at[srcv.at[i + _NB]], rows[b], gsem[b])
        return carry

    lax.fori_loop(0, _NCH // _NB - 1, body, 0)
    scats = []
    for b in range(_NB):
        i = _NCH - _NB + b
        pltpu.make_async_copy(y_hbm.at[srcv.at[i]], rows[b], gsem[b]).wait()
        scats.append(
            pltpu.async_copy(rows[b], acc.at[dstv.at[i]], ssem[b], add=True))
    for cp in scats:
        cp.wait()

    plsc.subcore_barrier()
    pltpu.sync_copy(acc.at[pl.ds(s * _RPT, _RPT)],
                    out_hbm.at[c].at[pl.ds(s * _RPT, _RPT)])


@jax.jit
def _sc_aggregate(y, src2d, dst2d):
    return pl.kernel(
        _agg_body,
        out_type=jax.ShapeDtypeStruct((_NC, _N, _H), jnp.float32),
        mesh=_sc_mesh(),
        scratch_types=[
            pltpu.VMEM((_NCH, _K), jnp.int32),
            pltpu.VMEM((_NCH, _K), jnp.int32),
        ] + [pltpu.VMEM((_K, _H), jnp.float32)] * _NB + [
            pltpu.VMEM((_ZR, _H), jnp.float32),
            pltpu.VMEM_SHARED((_N, _H), jnp.float32),
        ] + [pltpu.SemaphoreType.DMA] * (2 * _NB),
        compiler_params=pltpu.CompilerParams(use_tc_tiling_on_sc=False),
    )(y, src2d, dst2d)


def _gelu(x):
    return 0.5 * x * (1.0 + lax.erf(x / _SQRT2))


def _bn_gelu(gcn, g, be):
    m = jnp.mean(gcn, axis=0, keepdims=True)
    v = jnp.mean((gcn - m) ** 2, axis=0, keepdims=True)
    return _gelu((gcn - m) * lax.rsqrt(v + 1e-5) * g + be)


def _pre_body(x_ref, w_ref, degp_ref, y_ref, dinv_ref):
    deg = degp_ref[0, :, :1] + degp_ref[1, :, :1] + 1.0
    dinv = lax.rsqrt(deg)
    dinv_ref[...] = dinv
    y_ref[...] = dinv * jnp.dot(x_ref[...], w_ref[...],
                                preferred_element_type=jnp.float32)


def _mid_body(aggp_ref, y_ref, dinv_ref, b_ref, g_ref, be_ref, w_ref,
              out_ref):
    dinv = dinv_ref[...]
    gcn = dinv * (aggp_ref[0] + aggp_ref[1] + y_ref[...]) + b_ref[...]
    h = _bn_gelu(gcn, g_ref[...], be_ref[...])
    out_ref[...] = dinv * jnp.dot(h, w_ref[...],
                                  preferred_element_type=jnp.float32)


def _fin_body(aggp_ref, y_ref, dinv_ref, b_ref, g_ref, be_ref, batch_ref,
              lw1_ref, lb1_ref, lw2_ref, lb2_ref, out_ref):
    dinv = dinv_ref[...]
    gcn = dinv * (aggp_ref[0] + aggp_ref[1] + y_ref[...]) + b_ref[...]
    h = _bn_gelu(gcn, g_ref[...], be_ref[...])
    gids = lax.broadcasted_iota(jnp.int32, (_G, _N), 0)
    onehot = (batch_ref[...] == gids).astype(jnp.float32)
    sums = jnp.dot(onehot, h, preferred_element_type=jnp.float32)
    cnt = jnp.sum(onehot, axis=1, keepdims=True)
    pooled = sums / jnp.maximum(cnt, 1.0)
    o = pooled @ lw1_ref[...] + lb1_ref[...]
    o = jnp.where(o > 0, o, jnp.exp(jnp.minimum(o, 0.0)) - 1.0)
    out_ref[...] = o @ lw2_ref[...] + lb2_ref[...]


def kernel(x, edge_index, batch, edge_weight, W1, b1, g1, be1, W2, b2, g2,
           be2, W3, b3, g3, be3, lw1, lb1, lw2, lb2):
    src2d = edge_index[0].reshape(_E // _K, _K)
    dst2d = edge_index[1].reshape(_E // _K, _K)
    dst2d_deg = edge_index[1].reshape(_E // _DK, _DK)

    pre = pl.pallas_call(
        _pre_body, out_shape=[jax.ShapeDtypeStruct((_N, _H), jnp.float32),
                              jax.ShapeDtypeStruct((_N, 1), jnp.float32)])
    mid = pl.pallas_call(
        _mid_body, out_shape=jax.ShapeDtypeStruct((_N, _H), jnp.float32))
    fin = pl.pallas_call(
        _fin_body, out_shape=jax.ShapeDtypeStruct((_G, 1), jnp.float32))

    degp = _sc_degree(dst2d_deg)
    y1, dinv = pre(x, W1, degp)

    agg1 = _sc_aggregate(y1, src2d, dst2d)
    y2 = mid(agg1, y1, dinv, b1.reshape(1, _H), g1.reshape(1, _H),
             be1.reshape(1, _H), W2)
    agg2 = _sc_aggregate(y2, src2d, dst2d)
    y3 = mid(agg2, y2, dinv, b2.reshape(1, _H), g2.reshape(1, _H),
             be2.reshape(1, _H), W3)
    agg3 = _sc_aggregate(y3, src2d, dst2d)
    out = fin(agg3, y3, dinv, b3.reshape(1, _H), g3.reshape(1, _H),
              be3.reshape(1, _H), batch.reshape(1, _N),
              lw1, lb1.reshape(1, _H // 2), lw2, lb2.reshape(1, 1))
    return out
